# speedup vs baseline: 1.4048x; 1.4048x over previous
import functools

import jax
import jax.numpy as jnp
from jax.experimental import pallas as pl
from jax.experimental.pallas import tpu as pltpu


def _fused_body(eps_ref, x0_ref, inc_ref, w_ref, bias_ref, out_ref, x1_ref,
                *, n_steps):
    j = pl.program_id(0)
    bb = inc_ref[...].astype(jnp.bfloat16)
    x0 = x0_ref[...]
    x0_16 = x0.astype(jnp.bfloat16)

    x1 = jax.lax.dot_general(
        bb, x0_16, (((0,), (0,)), ((), ())),
        preferred_element_type=jnp.float32)
    x1_ref[...] = x1

    m_part = jax.lax.dot_general(
        bb, x1.astype(jnp.bfloat16), (((1,), (0,)), ((), ())),
        preferred_element_type=jnp.float32)

    @pl.when(j == 0)
    def _():
        out_ref[...] = m_part

    @pl.when(j > 0)
    def _():
        out_ref[...] += m_part

    @pl.when(j == n_steps - 1)
    def _():
        acc = out_ref[...] + (1.0 + eps_ref[0]) * x0
        out_ref[...] = jax.lax.dot_general(
            acc.astype(jnp.bfloat16), w_ref[...].astype(jnp.bfloat16),
            (((1,), (1,)), ((), ())),
            preferred_element_type=jnp.float32) + bias_ref[...]


@jax.jit
def kernel(x_0, incidence_1, W, b, eps):
    n, d = x_0.shape
    e = incidence_1.shape[1]
    blk_e = 128
    n_steps = e // blk_e

    bias = b.reshape(1, d)

    out, x1 = pl.pallas_call(
        functools.partial(_fused_body, n_steps=n_steps),
        grid=(n_steps,),
        in_specs=[
            pl.BlockSpec(memory_space=pltpu.SMEM),
            pl.BlockSpec((n, d), lambda j: (0, 0)),
            pl.BlockSpec((n, blk_e), lambda j: (0, j)),
            pl.BlockSpec((d, d), lambda j: (0, 0)),
            pl.BlockSpec((1, d), lambda j: (0, 0)),
        ],
        out_specs=[
            pl.BlockSpec((n, d), lambda j: (0, 0)),
            pl.BlockSpec((blk_e, d), lambda j: (j, 0)),
        ],
        out_shape=[
            jax.ShapeDtypeStruct((n, d), jnp.float32),
            jax.ShapeDtypeStruct((e, d), jnp.float32),
        ],
        compiler_params=pltpu.CompilerParams(
            dimension_semantics=("arbitrary",),
        ),
    )(eps, x_0, incidence_1, W, bias)
    return (out, x1)


# speedup vs baseline: 1.6283x; 1.1591x over previous
import functools

import jax
import jax.numpy as jnp
from jax.experimental import pallas as pl
from jax.experimental.pallas import tpu as pltpu


def _fused_body(eps_ref, x0_ref, inc_ref, w_ref, bias_ref, out_ref, x1_ref,
                *, n_steps):
    j = pl.program_id(0)
    bb = inc_ref[...].astype(jnp.bfloat16)

    x1 = jax.lax.dot_general(
        bb, x0_ref[...].astype(jnp.bfloat16), (((0,), (0,)), ((), ())),
        preferred_element_type=jnp.float32)
    x1_ref[...] = x1

    m_part = jax.lax.dot_general(
        bb, x1.astype(jnp.bfloat16), (((1,), (0,)), ((), ())),
        preferred_element_type=jnp.float32)

    @pl.when(j == 0)
    def _():
        out_ref[...] = m_part

    @pl.when(j > 0)
    def _():
        out_ref[...] += m_part

    @pl.when(j == n_steps - 1)
    def _():
        acc = out_ref[...] + (1.0 + eps_ref[0]) * x0_ref[...]
        out_ref[...] = jax.lax.dot_general(
            acc.astype(jnp.bfloat16), w_ref[...].astype(jnp.bfloat16),
            (((1,), (1,)), ((), ())),
            preferred_element_type=jnp.float32) + bias_ref[...]


@jax.jit
def kernel(x_0, incidence_1, W, b, eps):
    n, d = x_0.shape
    e = incidence_1.shape[1]
    blk_e = 256
    n_steps = e // blk_e

    bias = b.reshape(1, d)

    out, x1 = pl.pallas_call(
        functools.partial(_fused_body, n_steps=n_steps),
        grid=(n_steps,),
        in_specs=[
            pl.BlockSpec(memory_space=pltpu.SMEM),
            pl.BlockSpec((n, d), lambda j: (0, 0)),
            pl.BlockSpec((n, blk_e), lambda j: (0, j)),
            pl.BlockSpec((d, d), lambda j: (0, 0)),
            pl.BlockSpec((1, d), lambda j: (0, 0)),
        ],
        out_specs=[
            pl.BlockSpec((n, d), lambda j: (0, 0)),
            pl.BlockSpec((blk_e, d), lambda j: (j, 0)),
        ],
        out_shape=[
            jax.ShapeDtypeStruct((n, d), jnp.float32),
            jax.ShapeDtypeStruct((e, d), jnp.float32),
        ],
        compiler_params=pltpu.CompilerParams(
            dimension_semantics=("arbitrary",),
        ),
    )(eps, x_0, incidence_1, W, bias)
    return (out, x1)
